# baseline (speedup 1.0000x reference)
import functools

import jax
import jax.numpy as jnp
from jax import lax
from jax.experimental import pallas as pl
from jax.experimental.pallas import tpu as pltpu
from jax.experimental.pallas import tpu_sc as plsc

G = 10000
N = 320000
NC = 2
NS = 16
NW = NC * NS

B1 = 3200
PER_W = N // NW
CHUNK = 80
SW = 256
GPAD = 10240
BG = 2000


def _atom_mlp_body(sp_ref, f0_ref, f1_ref, f2_ref, idx_ref, m1_ref, b1_ref,
                   w2_ref, b2_ref, out_ref, cnt_ref):
    b = pl.program_id(0)

    @pl.when(b == 0)
    def _zero():
        cnt_ref[...] = jnp.zeros_like(cnt_ref)

    sp = sp_ref[...]
    rows = lax.broadcasted_iota(jnp.int32, (128, B1), 0)
    onehot_t = (rows == sp).astype(jnp.float32)
    a = jnp.concatenate(
        [onehot_t, f0_ref[...], f1_ref[...], f2_ref[...]], axis=0)
    h = lax.dot_general(a.astype(jnp.bfloat16), m1_ref[...],
                        (((0,), (0,)), ((), ())),
                        preferred_element_type=jnp.float32) + b1_ref[...]
    h = jnp.maximum(h, 0.0)
    out_ref[...] = (jnp.dot(h.astype(jnp.bfloat16), w2_ref[...],
                            preferred_element_type=jnp.float32)
                    + b2_ref[...])

    iv = idx_ref[...]
    lo = idx_ref[0, 0]
    hi = idx_ref[0, B1 - 1]
    base = (lo // SW) * SW
    n_strips = (hi - base) // SW + 1

    def strip(k, _):
        off = base + k * SW
        srows = off + lax.broadcasted_iota(jnp.int32, (SW, B1), 0)
        m = (srows == iv).astype(jnp.float32)
        cnt_ref[pl.ds(off, SW), :] += jnp.sum(m, axis=1, keepdims=True)
        return 0

    lax.fori_loop(0, n_strips, strip, 0)


def _atom_mlp(species_row, f0, f1, f2, idx_row, m1_aug, b1r, w2_t, b2r):
    grid = N // B1
    return pl.pallas_call(
        _atom_mlp_body,
        grid=(grid,),
        in_specs=[
            pl.BlockSpec((1, B1), lambda i: (0, i)),
            pl.BlockSpec((1, B1), lambda i: (0, i)),
            pl.BlockSpec((1, B1), lambda i: (0, i)),
            pl.BlockSpec((1, B1), lambda i: (0, i)),
            pl.BlockSpec((1, B1), lambda i: (0, i)),
            pl.BlockSpec((131, 128), lambda i: (0, 0)),
            pl.BlockSpec((1, 128), lambda i: (0, 0)),
            pl.BlockSpec((128, 128), lambda i: (0, 0)),
            pl.BlockSpec((1, 128), lambda i: (0, 0)),
        ],
        out_specs=[pl.BlockSpec((B1, 128), lambda i: (i, 0)),
                   pl.BlockSpec((GPAD, 1), lambda i: (0, 0))],
        out_shape=[jax.ShapeDtypeStruct((N, 128), jnp.float32),
                   jax.ShapeDtypeStruct((GPAD, 1), jnp.float32)],
        compiler_params=pltpu.CompilerParams(
            dimension_semantics=("arbitrary",)),
    )(species_row, f0, f1, f2, idx_row, m1_aug, b1r, w2_t, b2r)


def _seg_sum_sc(atom_feat, idx1d, zeros_p):
    mesh = plsc.VectorSubcoreMesh(core_axis_name="c", subcore_axis_name="s")
    n_iter = PER_W // CHUNK

    @functools.partial(
        pl.kernel, mesh=mesh,
        out_type=[jax.ShapeDtypeStruct((NC, G, 128), jnp.float32)],
        scratch_types=[
            pltpu.VMEM((CHUNK, 128), jnp.float32),
            pltpu.VMEM((CHUNK, 128), jnp.float32),
            pltpu.VMEM((CHUNK,), jnp.int32),
            pltpu.VMEM((CHUNK,), jnp.int32),
            pltpu.SemaphoreType.DMA,
            pltpu.SemaphoreType.DMA,
            pltpu.SemaphoreType.DMA,
            pltpu.SemaphoreType.DMA,
            pltpu.VMEM_SHARED((G, 128), jnp.float32),
        ],
    )
    def k(af_hbm, idx_hbm, zp_hbm, pp_hbm, rows0, rows1, iv0, iv1,
          sem0, sem1, ssem0, ssem1, shared_p):
        c = lax.axis_index("c")
        s = lax.axis_index("s")
        wid = c * NS + s
        base_w = wid * PER_W
        rows_b = (rows0, rows1)
        iv_b = (iv0, iv1)
        sem_b = (sem0, sem1)
        ssem_b = (ssem0, ssem1)

        @pl.when(s == 0)
        def _init():
            pltpu.sync_copy(zp_hbm, shared_p)

        plsc.subcore_barrier()

        def fire(i, slot):
            rb = base_w + i * CHUNK
            pltpu.async_copy(idx_hbm.at[pl.ds(rb, CHUNK)], iv_b[slot],
                             sem_b[slot])
            pltpu.async_copy(af_hbm.at[pl.ds(rb, CHUNK)], rows_b[slot],
                             sem_b[slot])

        def wait_loads(i, slot):
            rb = base_w + i * CHUNK
            pltpu.make_async_copy(idx_hbm.at[pl.ds(rb, CHUNK)], iv_b[slot],
                                  sem_b[slot]).wait()
            pltpu.make_async_copy(af_hbm.at[pl.ds(rb, CHUNK)], rows_b[slot],
                                  sem_b[slot]).wait()

        def fire_scatter(slot):
            pltpu.async_copy(rows_b[slot], shared_p.at[iv_b[slot]],
                             ssem_b[slot], add=True)

        def wait_scatter(slot):
            pltpu.make_async_copy(rows_b[slot], shared_p.at[iv_b[slot]],
                                  ssem_b[slot]).wait()

        fire(0, 0)
        fire(1, 1)

        def body(p, _):
            i0 = 2 * p
            wait_loads(i0, 0)
            fire_scatter(0)
            wait_loads(i0 + 1, 1)
            fire_scatter(1)
            wait_scatter(0)

            @pl.when(i0 + 2 < n_iter)
            def _f0():
                fire(i0 + 2, 0)

            wait_scatter(1)

            @pl.when(i0 + 3 < n_iter)
            def _f1():
                fire(i0 + 3, 1)

            return 0

        lax.fori_loop(0, n_iter // 2, body, 0)
        wait_loads(n_iter - 1, 0)
        pltpu.sync_copy(rows_b[0], shared_p.at[iv_b[0]], add=True)

        plsc.subcore_barrier()

        @pl.when(s == 0)
        def _flush():
            pltpu.sync_copy(shared_p, pp_hbm.at[c])

    return k(atom_feat, idx1d, zeros_p)[0]


def _graph_mlp_body(pp_ref, cc_ref, lat_ref, w3p_ref, w3l_ref, b3_ref,
                    w4_ref, b4_ref, mu_ref, lv_ref):
    pooled = pp_ref[0] + pp_ref[1]
    pooled = pooled / cc_ref[...]
    h2 = (jnp.dot(pooled, w3p_ref[...], preferred_element_type=jnp.float32)
          + jnp.dot(lat_ref[...], w3l_ref[...], preferred_element_type=jnp.float32)
          + b3_ref[...])
    h2 = jnp.maximum(h2, 0.0)
    params = (jnp.dot(h2, w4_ref[...], preferred_element_type=jnp.float32)
              + b4_ref[...])
    mu_ref[...] = params[:, :128]
    lv_ref[...] = params[:, 128:]


def _graph_mlp(pp, cc, lat9, w3p_t, w3l_t, b3r, w4_t, b4r):
    grid = G // BG
    return pl.pallas_call(
        _graph_mlp_body,
        grid=(grid,),
        in_specs=[
            pl.BlockSpec((NC, BG, 128), lambda i: (0, i, 0)),
            pl.BlockSpec((BG, 1), lambda i: (i, 0)),
            pl.BlockSpec((BG, 9), lambda i: (i, 0)),
            pl.BlockSpec((128, 128), lambda i: (0, 0)),
            pl.BlockSpec((9, 128), lambda i: (0, 0)),
            pl.BlockSpec((1, 128), lambda i: (0, 0)),
            pl.BlockSpec((128, 256), lambda i: (0, 0)),
            pl.BlockSpec((1, 256), lambda i: (0, 0)),
        ],
        out_specs=[pl.BlockSpec((BG, 128), lambda i: (i, 0)),
                   pl.BlockSpec((BG, 128), lambda i: (i, 0))],
        out_shape=[jax.ShapeDtypeStruct((G, 128), jnp.float32),
                   jax.ShapeDtypeStruct((G, 128), jnp.float32)],
        compiler_params=pltpu.CompilerParams(
            dimension_semantics=("parallel",)),
    )(pp, cc, lat9, w3p_t, w3l_t, b3r, w4_t, b4r)


def kernel(lattice, fracs, species, batch_indices, emb, W1, b1, W2, b2,
           W3, b3, W4, b4):
    idx1d = batch_indices.astype(jnp.int32)
    species_row = species.astype(jnp.int32).reshape(1, N)

    emb_p = jnp.zeros((128, 32), jnp.float32).at[:100].set(emb)
    m1_aug = jnp.concatenate(
        [emb_p @ W1[:, :32].T, W1[:, 32:].T], axis=0).astype(jnp.bfloat16)
    w2_t = W2.T.astype(jnp.bfloat16)
    f0 = fracs[:, 0].reshape(1, N)
    f1 = fracs[:, 1].reshape(1, N)
    f2 = fracs[:, 2].reshape(1, N)
    w3p_t = W3[:, :128].T
    w3l_t = W3[:, 128:].T
    w4_t = W4.T

    atom_feat, counts = _atom_mlp(species_row, f0, f1, f2, idx1d.reshape(1, N),
                                  m1_aug, b1.reshape(1, 128), w2_t,
                                  b2.reshape(1, 128))
    cc = counts[:G]

    zeros_p = jnp.zeros((G, 128), jnp.float32)
    pp = _seg_sum_sc(atom_feat, idx1d, zeros_p)

    lat9 = lattice.reshape(G, 9)
    mu, logvar = _graph_mlp(pp, cc, lat9, w3p_t, w3l_t,
                            b3.reshape(1, 128), w4_t, b4.reshape(1, 256))
    return (mu, logvar)
